# baseline (device time: 15181 ns/iter reference)
import jax
import jax.numpy as jnp
from jax import lax
from jax.experimental import pallas as pl
from jax.experimental.pallas import tpu as pltpu

N_DEV = 32


def kernel(x):
    m, n = x.shape

    def body(x_ref, out_ref, sbuf, rbuf, send_sems, recv_sems):
        my_i = lax.axis_index("i")
        v = x_ref[...]

        t = v
        h = m // 2
        while h >= 1:
            t = t[:h, :] * t[h : 2 * h, :]
            h //= 2
        sbuf[...] = t

        def pair_rdma(j):
            return pltpu.make_async_remote_copy(
                src_ref=sbuf,
                dst_ref=rbuf.at[my_i],
                send_sem=send_sems.at[j],
                recv_sem=recv_sems.at[my_i],
                device_id=(j,),
                device_id_type=pl.DeviceIdType.MESH,
            )

        for j in range(N_DEV):

            @pl.when(my_i < j)
            def _():
                pair_rdma(j).start()

        row = lax.broadcasted_iota(jnp.int32, (m, n), 0)
        ones_f = jnp.ones((m, n), v.dtype)
        shift = 1
        while shift < m:
            rolled = pltpu.roll(v, shift, 0)
            v = v * jnp.where(row >= shift, rolled, ones_f)
            shift *= 2

        def recv_rdma(j):
            return pltpu.make_async_remote_copy(
                src_ref=sbuf,
                dst_ref=rbuf.at[j],
                send_sem=send_sems.at[j],
                recv_sem=recv_sems.at[j],
                device_id=(j,),
                device_id_type=pl.DeviceIdType.MESH,
            )

        for j in range(N_DEV - 1):

            @pl.when(my_i > j)
            def _():
                recv_rdma(j).wait_recv()

        ones = jnp.ones((1, n), v.dtype)
        terms = [
            jnp.where(my_i > j, rbuf[j, :, :], ones) for j in range(N_DEV - 1)
        ]
        while len(terms) > 1:
            terms = [
                terms[k] * terms[k + 1] if k + 1 < len(terms) else terms[k]
                for k in range(0, len(terms), 2)
            ]
        prefix = terms[0]

        out_ref[...] = v * prefix

        for j in range(N_DEV):

            @pl.when(my_i < j)
            def _():
                pair_rdma(j).wait_send()

    return pl.pallas_call(
        body,
        out_shape=jax.ShapeDtypeStruct((m, n), x.dtype),
        in_specs=[pl.BlockSpec(memory_space=pltpu.VMEM)],
        out_specs=pl.BlockSpec(memory_space=pltpu.VMEM),
        scratch_shapes=[
            pltpu.VMEM((1, n), x.dtype),
            pltpu.VMEM((N_DEV, 1, n), x.dtype),
            pltpu.SemaphoreType.DMA((N_DEV,)),
            pltpu.SemaphoreType.DMA((N_DEV,)),
        ],
    )(x)


# device time: 12503 ns/iter; 1.2142x vs baseline; 1.2142x over previous
import jax
import jax.numpy as jnp
from jax import lax
from jax.experimental import pallas as pl
from jax.experimental.pallas import tpu as pltpu

N_DEV = 32


def kernel(x):
    m, n = x.shape

    def body(x_ref, out_ref, sbuf, rbuf, send_sems, recv_sems):
        my_i = lax.axis_index("i")
        v = x_ref[...]

        barrier_sem = pltpu.get_barrier_semaphore()

        def sig(k):
            pl.semaphore_signal(
                barrier_sem,
                inc=1,
                device_id=(lax.rem(my_i + k, N_DEV),),
                device_id_type=pl.DeviceIdType.MESH,
            )

        for k in range(1, 8):
            sig(k)

        t = v
        h = m // 2
        while h >= 1:
            t = t[:h, :] * t[h : 2 * h, :]
            h //= 2
        sbuf[...] = t

        row = lax.broadcasted_iota(jnp.int32, (m, n), 0)
        ones_f = jnp.ones((m, n), v.dtype)
        ksig = 8
        for shift in (1, 2, 4, 8):
            rolled = pltpu.roll(v, shift, 0)
            v = v * jnp.where(row >= shift, rolled, ones_f)
            for k in range(ksig, ksig + 6):
                if k < 32:
                    sig(k)
            ksig += 6

        pl.semaphore_wait(barrier_sem, N_DEV - 1)

        def pair_rdma(j):
            return pltpu.make_async_remote_copy(
                src_ref=sbuf,
                dst_ref=rbuf.at[my_i],
                send_sem=send_sems.at[j],
                recv_sem=recv_sems.at[my_i],
                device_id=(j,),
                device_id_type=pl.DeviceIdType.MESH,
            )

        for j in range(N_DEV):

            @pl.when(my_i < j)
            def _():
                pair_rdma(j).start()

        for shift in (16, 32, 64, 128, 256):
            rolled = pltpu.roll(v, shift, 0)
            v = v * jnp.where(row >= shift, rolled, ones_f)

        def recv_rdma(j):
            return pltpu.make_async_remote_copy(
                src_ref=sbuf,
                dst_ref=rbuf.at[j],
                send_sem=send_sems.at[j],
                recv_sem=recv_sems.at[j],
                device_id=(j,),
                device_id_type=pl.DeviceIdType.MESH,
            )

        ones = jnp.ones((1, n), v.dtype)
        prefix = ones
        for j in range(N_DEV - 2, -1, -1):

            @pl.when(my_i > j)
            def _():
                recv_rdma(j).wait_recv()

            prefix = prefix * jnp.where(my_i > j, rbuf[j, :, :], ones)

        out_ref[...] = v * prefix

        for j in range(N_DEV):

            @pl.when(my_i < j)
            def _():
                pair_rdma(j).wait_send()

    return pl.pallas_call(
        body,
        out_shape=jax.ShapeDtypeStruct((m, n), x.dtype),
        in_specs=[pl.BlockSpec(memory_space=pltpu.VMEM)],
        out_specs=pl.BlockSpec(memory_space=pltpu.VMEM),
        scratch_shapes=[
            pltpu.VMEM((1, n), x.dtype),
            pltpu.VMEM((N_DEV, 1, n), x.dtype),
            pltpu.SemaphoreType.DMA((N_DEV,)),
            pltpu.SemaphoreType.DMA((N_DEV,)),
        ],
        compiler_params=pltpu.CompilerParams(collective_id=0),
    )(x)
